# Initial kernel scaffold; baseline (speedup 1.0000x reference)
import jax
import jax.numpy as jnp
from jax.experimental import pallas as pl


def kernel(data, mu, S_inv, n, cluster_labels):
    raise NotImplementedError("write your pallas kernel here")


# speedup vs baseline: 2.4259x; 2.4259x over previous
import jax
import jax.numpy as jnp
from jax import lax
from jax.experimental import pallas as pl
from jax.experimental.pallas import tpu as pltpu

B = 1024
C = 512
D = 64
NUM_CLASSES = 10
KAPPA_N = 10.0
CB = 8


def _egaussp_kernel(data_ref, mu_ref, sinv_ref, n_ref, labels_ref,
                    scores_ref, preds_ref, clusters_ref, gamma_ref):
    x = data_ref[:]

    def step(i, carry):
        mu_b = mu_ref[pl.ds(i * CB, CB), :]
        s_b = sinv_ref[pl.ds(i * CB, CB), :, :]
        diff = x[None, :, :] - mu_b[:, None, :]
        tmp = lax.dot_general(
            diff, s_b, (((2,), (1,)), ((0,), (0,))),
            preferred_element_type=jnp.float32)
        d2 = jnp.sum(tmp * diff, axis=2)
        g = jnp.exp(-0.5 * d2)
        mask = n_ref[pl.ds(i * CB, CB), :] >= KAPPA_N
        g = jnp.where(mask, g, 0.0)
        gamma_ref[pl.ds(i * CB, CB), :] = g
        return carry

    lax.fori_loop(0, C // CB, step, 0, unroll=False)

    G = gamma_ref[:]
    denom = jnp.sum(G, axis=0) + 1e-12
    raw = lax.dot_general(
        G, labels_ref[:], (((0,), (0,)), ((), ())),
        preferred_element_type=jnp.float32)
    scores = raw / denom[:, None]
    scores_ref[:] = scores

    it = lax.broadcasted_iota(jnp.int32, scores.shape, 1)
    mx = jnp.max(scores, axis=1, keepdims=True)
    preds_ref[:] = jnp.min(jnp.where(scores == mx, it, NUM_CLASSES),
                           axis=1, keepdims=True)

    itc = lax.broadcasted_iota(jnp.int32, G.shape, 0)
    mxc = jnp.max(G, axis=0, keepdims=True)
    clusters_ref[:] = jnp.min(jnp.where(G == mxc, itc, C),
                              axis=0, keepdims=True)


def kernel(data, mu, S_inv, n, cluster_labels):
    n2 = n.reshape(C, 1)
    scores, preds, clusters = pl.pallas_call(
        _egaussp_kernel,
        out_shape=[
            jax.ShapeDtypeStruct((B, NUM_CLASSES), jnp.float32),
            jax.ShapeDtypeStruct((B, 1), jnp.int32),
            jax.ShapeDtypeStruct((1, B), jnp.int32),
        ],
        scratch_shapes=[pltpu.VMEM((C, B), jnp.float32)],
    )(data, mu, S_inv, n2, cluster_labels)
    return (scores, preds[:, 0], clusters[0, :])


# speedup vs baseline: 2.5815x; 1.0641x over previous
import jax
import jax.numpy as jnp
from jax import lax
from jax.experimental import pallas as pl
from jax.experimental.pallas import tpu as pltpu

B = 1024
C = 512
D = 64
NUM_CLASSES = 10
KAPPA_N = 10.0
CB = 8


def _egaussp_kernel(data_ref, mu_ref, sinv_ref, n_ref, labels_ref,
                    scores_ref, preds_ref, clusters_ref, gamma_ref):
    x = data_ref[:]

    def step(i, carry):
        mu_b = mu_ref[pl.ds(i * CB, CB), :]
        s_b = sinv_ref[pl.ds(i * CB, CB), :, :]
        diff = x[None, :, :] - mu_b[:, None, :]
        tmp = lax.dot_general(
            diff, s_b, (((2,), (1,)), ((0,), (0,))),
            preferred_element_type=jnp.float32)
        d2 = jnp.sum(tmp * diff, axis=2)
        g = jnp.exp(-0.5 * d2)
        mask = n_ref[pl.ds(i * CB, CB), :] >= KAPPA_N
        g = jnp.where(mask, g, 0.0)
        gamma_ref[pl.ds(i * CB, CB), :] = g
        return carry

    lax.fori_loop(0, C // CB, step, 0, unroll=2)

    G = gamma_ref[:]
    denom = jnp.sum(G, axis=0) + 1e-12
    raw = lax.dot_general(
        G, labels_ref[:], (((0,), (0,)), ((), ())),
        preferred_element_type=jnp.float32)
    scores = raw / denom[:, None]
    scores_ref[:] = scores

    it = lax.broadcasted_iota(jnp.int32, scores.shape, 1)
    mx = jnp.max(scores, axis=1, keepdims=True)
    preds_ref[:] = jnp.min(jnp.where(scores == mx, it, NUM_CLASSES),
                           axis=1, keepdims=True)

    itc = lax.broadcasted_iota(jnp.int32, G.shape, 0)
    mxc = jnp.max(G, axis=0, keepdims=True)
    clusters_ref[:] = jnp.min(jnp.where(G == mxc, itc, C),
                              axis=0, keepdims=True)


def kernel(data, mu, S_inv, n, cluster_labels):
    n2 = n.reshape(C, 1)
    scores, preds, clusters = pl.pallas_call(
        _egaussp_kernel,
        out_shape=[
            jax.ShapeDtypeStruct((B, NUM_CLASSES), jnp.float32),
            jax.ShapeDtypeStruct((B, 1), jnp.int32),
            jax.ShapeDtypeStruct((1, B), jnp.int32),
        ],
        scratch_shapes=[pltpu.VMEM((C, B), jnp.float32)],
    )(data, mu, S_inv, n2, cluster_labels)
    return (scores, preds[:, 0], clusters[0, :])


# speedup vs baseline: 2.6723x; 1.0352x over previous
import jax
import jax.numpy as jnp
from jax import lax
from jax.experimental import pallas as pl
from jax.experimental.pallas import tpu as pltpu

B = 1024
C = 512
D = 64
NUM_CLASSES = 10
KAPPA_N = 10.0
CB = 16


def _egaussp_kernel(data_ref, mu_ref, sinv_ref, n_ref, labels_ref,
                    scores_ref, preds_ref, clusters_ref, gamma_ref):
    x = data_ref[:]

    def step(i, carry):
        mu_b = mu_ref[pl.ds(i * CB, CB), :]
        s_b = sinv_ref[pl.ds(i * CB, CB), :, :]
        diff = x[None, :, :] - mu_b[:, None, :]
        tmp = lax.dot_general(
            diff, s_b, (((2,), (1,)), ((0,), (0,))),
            preferred_element_type=jnp.float32)
        d2 = jnp.sum(tmp * diff, axis=2)
        g = jnp.exp(-0.5 * d2)
        mask = n_ref[pl.ds(i * CB, CB), :] >= KAPPA_N
        g = jnp.where(mask, g, 0.0)
        gamma_ref[pl.ds(i * CB, CB), :] = g
        return carry

    lax.fori_loop(0, C // CB, step, 0, unroll=2)

    G = gamma_ref[:]
    denom = jnp.sum(G, axis=0) + 1e-12
    raw = lax.dot_general(
        G, labels_ref[:], (((0,), (0,)), ((), ())),
        preferred_element_type=jnp.float32)
    scores = raw / denom[:, None]
    scores_ref[:] = scores

    it = lax.broadcasted_iota(jnp.int32, scores.shape, 1)
    mx = jnp.max(scores, axis=1, keepdims=True)
    preds_ref[:] = jnp.min(jnp.where(scores == mx, it, NUM_CLASSES),
                           axis=1, keepdims=True)

    itc = lax.broadcasted_iota(jnp.int32, G.shape, 0)
    mxc = jnp.max(G, axis=0, keepdims=True)
    clusters_ref[:] = jnp.min(jnp.where(G == mxc, itc, C),
                              axis=0, keepdims=True)


def kernel(data, mu, S_inv, n, cluster_labels):
    n2 = n.reshape(C, 1)
    scores, preds, clusters = pl.pallas_call(
        _egaussp_kernel,
        out_shape=[
            jax.ShapeDtypeStruct((B, NUM_CLASSES), jnp.float32),
            jax.ShapeDtypeStruct((B, 1), jnp.int32),
            jax.ShapeDtypeStruct((1, B), jnp.int32),
        ],
        scratch_shapes=[pltpu.VMEM((C, B), jnp.float32)],
    )(data, mu, S_inv, n2, cluster_labels)
    return (scores, preds[:, 0], clusters[0, :])
